# baseline (device time: 17924 ns/iter reference)
import jax
import jax.numpy as jnp
from jax import lax
from jax.experimental import pallas as pl
from jax.experimental.pallas import tpu as pltpu

N_DEV = 4


def kernel(x, w_mat):
    m_per, k = x.shape
    n_per = w_mat.shape[1]
    M = N_DEV * m_per
    h2 = m_per // 2

    def body(x_ref, w_ref, out_ref, comm_ref, send_sems, recv_sems):
        my = lax.axis_index("i")
        left = (my - 1) % N_DEV
        right = (my + 1) % N_DEV

        barrier_sem = pltpu.get_barrier_semaphore()
        for nbr in [left, right]:
            pl.semaphore_signal(
                barrier_sem, inc=1,
                device_id=(nbr,), device_id_type=pl.DeviceIdType.MESH,
            )
        pl.semaphore_wait(barrier_sem, 2)

        x_bf16 = x_ref[...].astype(jnp.bfloat16)
        comm_ref[0] = x_bf16

        def remote_copy(src, dst, sem, target):
            return pltpu.make_async_remote_copy(
                src_ref=src, dst_ref=dst,
                send_sem=send_sems.at[sem], recv_sem=recv_sems.at[sem],
                device_id=(target,), device_id_type=pl.DeviceIdType.MESH,
            )

        lo = pl.ds(0, h2)
        hi = pl.ds(h2, h2)

        cw1a = remote_copy(comm_ref.at[0, lo, :], comm_ref.at[1, lo, :], 0, right)
        ccw1a = remote_copy(comm_ref.at[0, hi, :], comm_ref.at[3, hi, :], 1, left)
        cw1a.start()
        ccw1a.start()
        cw1b = remote_copy(comm_ref.at[0, hi, :], comm_ref.at[1, hi, :], 2, right)
        ccw1b = remote_copy(comm_ref.at[0, lo, :], comm_ref.at[3, lo, :], 3, left)
        cw1b.start()
        ccw1b.start()

        w = w_ref[...].astype(jnp.bfloat16)

        def silu_store(row_start, n_rows, chunk_bf16):
            y = jnp.dot(chunk_bf16, w, preferred_element_type=jnp.float32)
            out_ref[pl.ds(row_start, n_rows), :] = y * jax.nn.sigmoid(y)

        silu_store(my * m_per, m_per, x_bf16)

        h4 = h2 // 2
        q = [pl.ds(i * h4, h4) for i in range(4)]
        cw1a.wait_recv()
        cw2a = remote_copy(comm_ref.at[1, q[0], :], comm_ref.at[2, q[0], :], 4, right)
        cw2b = remote_copy(comm_ref.at[1, q[1], :], comm_ref.at[2, q[1], :], 5, right)
        cw2a.start()
        cw2b.start()
        ccw1a.wait_recv()
        ccw2a = remote_copy(comm_ref.at[3, q[2], :], comm_ref.at[2, q[2], :], 6, left)
        ccw2b = remote_copy(comm_ref.at[3, q[3], :], comm_ref.at[2, q[3], :], 7, left)
        ccw2a.start()
        ccw2b.start()

        cw1b.wait_recv()
        silu_store(left * m_per, m_per, comm_ref[1])
        ccw1b.wait_recv()
        silu_store(right * m_per, m_per, comm_ref[3])

        opp = (my + 2) % N_DEV
        for i, r in enumerate((cw2a, cw2b, ccw2a, ccw2b)):
            r.wait_recv()
            silu_store(opp * m_per + i * h4, h4, comm_ref[2, q[i], :])

        for r in (cw1a, ccw1a, cw1b, ccw1b, cw2a, cw2b, ccw2a, ccw2b):
            r.wait_send()

    return pl.pallas_call(
        body,
        out_shape=jax.ShapeDtypeStruct((M, n_per), jnp.float32),
        in_specs=[
            pl.BlockSpec(memory_space=pltpu.VMEM),
            pl.BlockSpec(memory_space=pltpu.VMEM),
        ],
        out_specs=pl.BlockSpec(memory_space=pltpu.VMEM),
        scratch_shapes=[
            pltpu.VMEM((N_DEV, m_per, k), jnp.bfloat16),
            pltpu.SemaphoreType.DMA((8,)),
            pltpu.SemaphoreType.DMA((8,)),
        ],
        compiler_params=pltpu.CompilerParams(collective_id=0),
    )(x, w_mat)


# device time: 16507 ns/iter; 1.0858x vs baseline; 1.0858x over previous
import jax
import jax.numpy as jnp
from jax import lax
from jax.experimental import pallas as pl
from jax.experimental.pallas import tpu as pltpu

N_DEV = 4


def kernel(x, w_mat):
    m_per, k = x.shape
    n_per = w_mat.shape[1]
    M = N_DEV * m_per
    h2 = m_per // 2

    def body(x_ref, w_ref, out_ref, comm_ref, send_sems, recv_sems):
        my = lax.axis_index("i")
        left = (my - 1) % N_DEV
        right = (my + 1) % N_DEV

        barrier_sem = pltpu.get_barrier_semaphore()
        for nbr in [left, right]:
            pl.semaphore_signal(
                barrier_sem, inc=1,
                device_id=(nbr,), device_id_type=pl.DeviceIdType.MESH,
            )
        pl.semaphore_wait(barrier_sem, 2)

        x_bf16 = x_ref[...].astype(jnp.bfloat16)
        comm_ref[0] = x_bf16

        def remote_copy(src, dst, sem, target):
            return pltpu.make_async_remote_copy(
                src_ref=src, dst_ref=dst,
                send_sem=send_sems.at[sem], recv_sem=recv_sems.at[sem],
                device_id=(target,), device_id_type=pl.DeviceIdType.MESH,
            )

        lo = pl.ds(0, h2)
        hi = pl.ds(h2, h2)

        cw1a = remote_copy(comm_ref.at[0, lo, :], comm_ref.at[1, lo, :], 0, right)
        ccw1a = remote_copy(comm_ref.at[0, hi, :], comm_ref.at[3, hi, :], 1, left)
        cw1a.start()
        ccw1a.start()
        cw1b = remote_copy(comm_ref.at[0, hi, :], comm_ref.at[1, hi, :], 2, right)
        ccw1b = remote_copy(comm_ref.at[0, lo, :], comm_ref.at[3, lo, :], 3, left)
        cw1b.start()
        ccw1b.start()

        w = w_ref[...].astype(jnp.bfloat16)

        def silu_store(row_start, n_rows, chunk_bf16):
            y = jnp.dot(chunk_bf16, w, preferred_element_type=jnp.float32)
            out_ref[pl.ds(row_start, n_rows), :] = y * jax.nn.sigmoid(y)

        silu_store(my * m_per, m_per, x_bf16)

        cw1a.wait_recv()
        cw2 = remote_copy(comm_ref.at[1, lo, :], comm_ref.at[2, lo, :], 4, right)
        cw2.start()
        ccw1a.wait_recv()
        ccw2 = remote_copy(comm_ref.at[3, hi, :], comm_ref.at[2, hi, :], 5, left)
        ccw2.start()

        cw1b.wait_recv()
        silu_store(left * m_per, m_per, comm_ref[1])
        ccw1b.wait_recv()
        silu_store(right * m_per, m_per, comm_ref[3])

        opp = (my + 2) % N_DEV
        cw2.wait_recv()
        silu_store(opp * m_per, h2, comm_ref[2, lo, :])
        ccw2.wait_recv()
        silu_store(opp * m_per + h2, h2, comm_ref[2, hi, :])

        for r in (cw1a, ccw1a, cw1b, ccw1b, cw2, ccw2):
            r.wait_send()

    return pl.pallas_call(
        body,
        out_shape=jax.ShapeDtypeStruct((M, n_per), jnp.float32),
        in_specs=[
            pl.BlockSpec(memory_space=pltpu.VMEM),
            pl.BlockSpec(memory_space=pltpu.VMEM),
        ],
        out_specs=pl.BlockSpec(memory_space=pltpu.VMEM),
        scratch_shapes=[
            pltpu.VMEM((N_DEV, m_per, k), jnp.bfloat16),
            pltpu.SemaphoreType.DMA((6,)),
            pltpu.SemaphoreType.DMA((6,)),
        ],
        compiler_params=pltpu.CompilerParams(collective_id=0),
    )(x, w_mat)
